# baseline (speedup 1.0000x reference)
import math

import jax
import jax.numpy as jnp
from jax.experimental import pallas as pl


def _hconv(x, hedge_index, hedge_attr, W):
    node_idx = hedge_index[0]
    edge_idx = hedge_index[1]
    n_nodes = x.shape[0]
    n_edges = hedge_attr.shape[0]
    xw = x @ W
    hw = hedge_attr @ W
    x_j = jnp.take(xw, node_idx, axis=0)
    h_e = jnp.take(hw, edge_idx, axis=0)
    alpha = jax.nn.sigmoid(jnp.sum(x_j * h_e, axis=-1, keepdims=True) / math.sqrt(x.shape[1]))
    Bdeg = jnp.maximum(jnp.zeros((n_edges,), xw.dtype).at[edge_idx].add(1.0), 1.0)
    Ddeg = jnp.maximum(jnp.zeros((n_nodes,), xw.dtype).at[node_idx].add(1.0), 1.0)
    m_e = jax.ops.segment_sum(x_j * alpha, edge_idx, num_segments=n_edges) / Bdeg[:, None]
    out = jax.ops.segment_sum(jnp.take(m_e, edge_idx, axis=0), node_idx, num_segments=n_nodes) / Ddeg[:, None]
    return out


def _head_body(stu_v_ref, exer_v_ref, kn_v_ref, Wfc4_ref, bfc4_ref, Wfc5_ref,
               bfc5_ref, Wfc3_ref, bfc3_ref, Wfc7_ref, bfc7_ref, out_ref):
    stu_v = stu_v_ref[...]
    exer_v = exer_v_ref[...]
    kn_v = kn_v_ref[...]
    xs_in = jnp.concatenate([stu_v, kn_v], axis=-1)
    xe_in = jnp.concatenate([exer_v, kn_v], axis=-1)
    xs = jnp.tanh(xs_in @ Wfc4_ref[...] + bfc4_ref[...])
    xe = jnp.tanh(xe_in @ Wfc5_ref[...] + bfc5_ref[...])
    h = jax.nn.relu((xs - xe) @ Wfc3_ref[...] + bfc3_ref[...])
    out_ref[...] = jax.nn.sigmoid(h @ Wfc7_ref[...] + bfc7_ref[...])


def kernel(stu_id, exer_id, kn_r, hidx_sk, hidx_ek, stu_table, exer_table, kn_table,
           Ws1, Ws2, Wks1, Wks2, We1, We2, Wke1, Wke2,
           Wfc4, bfc4, Wfc5, bfc5, Wfc3, bfc3, Wfc7, bfc7):
    s = _hconv(stu_table, hidx_sk, kn_table, Ws1)
    s = _hconv(s, hidx_sk, kn_table, Ws2)
    s = jax.nn.relu(s)
    rev_sk = hidx_sk[::-1]
    k1 = _hconv(kn_table, rev_sk, s, Wks1)
    k1 = _hconv(k1, rev_sk, s, Wks2)
    k1 = jax.nn.relu(k1)
    e = _hconv(exer_table, hidx_ek, kn_table, We1)
    e = _hconv(e, hidx_ek, kn_table, We2)
    e = jax.nn.relu(e)
    rev_ek = hidx_ek[::-1]
    k2 = _hconv(kn_table, rev_ek, e, Wke1)
    k2 = _hconv(k2, rev_ek, e, Wke2)
    k2 = jax.nn.relu(k2)
    k = 0.5 * (k1 + k2)

    stu_v = jnp.take(s, stu_id, axis=0)
    exer_v = jnp.take(e, exer_id, axis=0)
    kn_v = (kn_r @ k) / (jnp.sum(kn_r, axis=-1, keepdims=True) + 1e-8)

    B = stu_v.shape[0]
    D = stu_v.shape[1]
    BLK = 512
    grid = (B // BLK,)
    out = pl.pallas_call(
        _head_body,
        out_shape=jax.ShapeDtypeStruct((B, 1), jnp.float32),
        grid=grid,
        in_specs=[
            pl.BlockSpec((BLK, D), lambda i: (i, 0)),
            pl.BlockSpec((BLK, D), lambda i: (i, 0)),
            pl.BlockSpec((BLK, D), lambda i: (i, 0)),
            pl.BlockSpec(Wfc4.shape, lambda i: (0, 0)),
            pl.BlockSpec(bfc4.shape, lambda i: (0,)),
            pl.BlockSpec(Wfc5.shape, lambda i: (0, 0)),
            pl.BlockSpec(bfc5.shape, lambda i: (0,)),
            pl.BlockSpec(Wfc3.shape, lambda i: (0, 0)),
            pl.BlockSpec(bfc3.shape, lambda i: (0,)),
            pl.BlockSpec(Wfc7.shape, lambda i: (0, 0)),
            pl.BlockSpec(bfc7.shape, lambda i: (0,)),
        ],
        out_specs=pl.BlockSpec((BLK, 1), lambda i: (i, 0)),
    )(stu_v, exer_v, kn_v, Wfc4, bfc4, Wfc5, bfc5, Wfc3, bfc3, Wfc7, bfc7)
    return out


# speedup vs baseline: 1.5813x; 1.5813x over previous
import functools
import math

import jax
import jax.numpy as jnp
from jax import lax
from jax.experimental import pallas as pl
from jax.experimental.pallas import tpu as pltpu
from jax.experimental.pallas import tpu_sc as plsc

NC = 2
NS = 16
NW = NC * NS
L = 16
D = 128
CH = 128
K_NUM = 2000
KPAD = 2048
RSQ = 1.0 / math.sqrt(D)

_mesh = functools.partial(
    plsc.VectorSubcoreMesh, core_axis_name="c", subcore_axis_name="s",
    num_cores=NC, num_subcores=NS)
_SC_PARAMS = pltpu.CompilerParams(needs_layout_passes=False)


def _f32(shape):
    return jax.ShapeDtypeStruct(shape, jnp.float32)


def _fill_ones(ref):
    for i in range(ref.shape[0] // L):
        ref[pl.ds(i * L, L)] = jnp.full((L,), 1.0, jnp.float32)


def _hsum_splat(tmp_ref, v):
    lanes = lax.iota(jnp.int32, L)
    for sh in (8, 4, 2, 1):
        tmp_ref[...] = v
        v = v + plsc.load_gather(tmp_ref, [lanes ^ sh])
    return v


def _split2k(sid, copy_fn):
    @pl.when(sid < 15)
    def _():
        copy_fn(sid * CH, CH)

    @pl.when(sid == 15)
    def _():
        copy_fn(15 * CH, 80)


def _make_a_kernel(nnz, nx, alpha):
    def body(*refs):
        if alpha:
            (x_hbm, g_hbm, nidx, eidx, z2d, z1d, mpart, cpart,
             m_sp, cnt_sp, nbuf, ebuf, xbuf, gbuf, abuf, ones_v,
             tmp16, sem1, sem2) = refs
        else:
            (x_hbm, nidx, eidx, z2d, z1d, mpart, cpart,
             m_sp, cnt_sp, nbuf, ebuf, xbuf, ones_v, sem1) = refs
        cid = lax.axis_index("c")
        sid = lax.axis_index("s")
        w = sid * NC + cid
        def _zm(off, sz):
            pltpu.sync_copy(z2d.at[pl.ds(0, sz)], m_sp.at[pl.ds(off, sz)])

        _split2k(sid, _zm)
        pltpu.sync_copy(z1d, cnt_sp.at[pl.ds(sid * CH, CH)])
        _fill_ones(ones_v)
        plsc.subcore_barrier()

        nchunks = nnz // CH
        trips = (nchunks - w + NW - 1) // NW

        def chunk_body(j, carry):
            base = (w + j * NW) * CH
            pltpu.sync_copy(nidx.at[pl.ds(base, CH)], nbuf)
            pltpu.sync_copy(eidx.at[pl.ds(base, CH)], ebuf)
            cpx = pltpu.async_copy(x_hbm.at[nbuf], xbuf, sem1)
            if alpha:
                cpg = pltpu.async_copy(g_hbm.at[ebuf], gbuf, sem2)
            cpx.wait()
            if alpha:
                cpg.wait()

                def row_body(r, c2):
                    acc = jnp.zeros((L,), jnp.float32)
                    xr = []
                    for g in range(8):
                        xv = xbuf[r, pl.ds(g * L, L)]
                        xr.append(xv)
                        acc = acc + xv * gbuf[r, pl.ds(g * L, L)]
                    z = _hsum_splat(tmp16, acc)
                    av = 1.0 / (1.0 + jnp.exp(-z * RSQ))
                    for g in range(8):
                        abuf[r, pl.ds(g * L, L)] = xr[g] * av
                    return c2

                lax.fori_loop(0, CH, row_body, 0)
                src = abuf
            else:
                src = xbuf
            pltpu.sync_copy(src, m_sp.at[ebuf], add=True)
            pltpu.sync_copy(ones_v, cnt_sp.at[ebuf], add=True)
            return carry

        lax.fori_loop(0, trips, chunk_body, 0)
        plsc.subcore_barrier()

        def _wm(off, sz):
            pltpu.sync_copy(m_sp.at[pl.ds(off, sz)],
                            mpart.at[cid, pl.ds(off, sz)])

        _split2k(sid, _wm)
        pltpu.sync_copy(cnt_sp.at[pl.ds(sid * CH, CH)],
                        cpart.at[cid, pl.ds(sid * CH, CH)])

    scratch = [
        pltpu.VMEM_SHARED((K_NUM, D), jnp.float32),
    ]
    scratch += [
        pltpu.VMEM_SHARED((KPAD,), jnp.float32),
        pltpu.VMEM((CH,), jnp.int32),
        pltpu.VMEM((CH,), jnp.int32),
        pltpu.VMEM((CH, D), jnp.float32),
    ]
    if alpha:
        scratch += [
            pltpu.VMEM((CH, D), jnp.float32),
            pltpu.VMEM((CH, D), jnp.float32),
        ]
    scratch += [
        pltpu.VMEM((CH,), jnp.float32),
    ]
    if alpha:
        scratch.append(pltpu.VMEM((L,), jnp.float32))
    scratch.append(pltpu.SemaphoreType.DMA)
    if alpha:
        scratch.append(pltpu.SemaphoreType.DMA)

    return pl.kernel(
        body,
        out_type=(_f32((NC, K_NUM, D)), _f32((NC, KPAD))),
        mesh=_mesh(),
        compiler_params=_SC_PARAMS,
        scratch_types=scratch,
    )


def _make_b_kernel(nnz, npad, rchunk, mode, relu):
    rr = rchunk
    racc = rr + CH
    nch = npad // rr
    qmax = (nch + NC - 1) // NC
    rm = mode == "rm"
    fh = 64 if rm else 128
    fmask = 2 * fh - 1
    fshift = fh.bit_length() - 1

    def body(*refs):
        if rm:
            (s_hbm, t_hbm, nidx, eidx, z2d, z1d, out_hbm,
             acc_sp, cnt_sp, nbuf, ebuf, pend_nl, pend_ng, pend_e,
             rows_v, trows_v, abuf, wbuf, cbuf, ones_v, tmp16,
             sem1, sem2) = refs
        else:
            (m_hbm, nidx, eidx, z2d, z1d, out_hbm,
             acc_sp, cnt_sp, nbuf, ebuf, pend_nl, pend_e,
             rows_v, wbuf, cbuf, ones_v, tmp16, sem1) = refs
        cid = lax.axis_index("c")
        sid = lax.axis_index("s")
        small_hbm = t_hbm if rm else m_hbm
        _fill_ones(ones_v)

        def flush(par):
            idx_e = pend_e.at[par]
            idx_nl = pend_nl.at[par]
            if rm:
                cp1 = pltpu.async_copy(s_hbm.at[pend_ng.at[par]], rows_v,
                                       sem1)
                cp2 = pltpu.async_copy(t_hbm.at[idx_e], trows_v, sem2)
                cp1.wait()
                cp2.wait()

                def row_body(r, c2):
                    acc = jnp.zeros((L,), jnp.float32)
                    tw = []
                    for g in range(8):
                        acc = acc + (rows_v[r, pl.ds(g * L, L)] *
                                     trows_v[r, pl.ds(g * L, L)])
                        tw.append(trows_v[r, pl.ds(D + g * L, L)])
                    z = _hsum_splat(tmp16, acc)
                    av = 1.0 / (1.0 + jnp.exp(-z * RSQ))
                    for g in range(8):
                        abuf[r, pl.ds(g * L, L)] = tw[g] * av
                    return c2

                lax.fori_loop(0, fh, row_body, 0)
                pltpu.sync_copy(abuf, acc_sp.at[idx_nl], add=True)
            else:
                cp = pltpu.async_copy(m_hbm.at[idx_e], rows_v, sem1)
                cp.wait()
                pltpu.sync_copy(rows_v, acc_sp.at[idx_nl], add=True)
            pltpu.sync_copy(ones_v, cnt_sp.at[idx_nl], add=True)

        nblk_acc = racc // CH
        nblk_real = rr // CH
        scan_chunks = nnz // CH

        def q_body(q, qcarry):
            ch = cid + NC * q

            @pl.when(ch < nch)
            def _do_chunk():
                lo = ch * rr
                hi = lo + rr

                def zero_body(bj, c2):
                    b = sid + bj * NS
                    pltpu.sync_copy(z2d, acc_sp.at[pl.ds(b * CH, CH)])
                    pltpu.sync_copy(z1d, cnt_sp.at[pl.ds(b * CH, CH)])
                    return c2

                lax.fori_loop(0, (nblk_acc - sid + NS - 1) // NS,
                              zero_body, 0)
                plsc.subcore_barrier()

                def scan_body(j, carry):
                    pc, fl = carry
                    base = (sid + j * NS) * CH
                    pltpu.sync_copy(nidx.at[pl.ds(base, CH)], nbuf)
                    pltpu.sync_copy(eidx.at[pl.ds(base, CH)], ebuf)
                    for v in range(8):
                        nv = nbuf[pl.ds(v * L, L)]
                        ev = ebuf[pl.ds(v * L, L)]
                        within = (nv >= lo) & (nv < hi)
                        wi = within.astype(jnp.int32)
                        cs = plsc.cumsum(wi)
                        pos = (jnp.full((L,), pc, jnp.int32) + cs - 1) & fmask
                        hi_i = pos >> fshift
                        lo_i = pos & (fh - 1)
                        plsc.store_scatter(pend_nl, [hi_i, lo_i], nv - lo,
                                           mask=within)
                        if rm:
                            plsc.store_scatter(pend_ng, [hi_i, lo_i], nv,
                                               mask=within)
                        plsc.store_scatter(pend_e, [hi_i, lo_i], ev,
                                           mask=within)
                        pc = pc + cs[L - 1]
                        do = (pc - fl) >= fh

                        @pl.when(do)
                        def _():
                            flush((fl // fh) & 1)

                        fl = jnp.where(do, fl + fh, fl)
                    return pc, fl

                pc, fl = lax.fori_loop(
                    0, (scan_chunks - sid + NS - 1) // NS, scan_body,
                    (jnp.int32(0), jnp.int32(0)))

                @pl.when(pc > fl)
                def _drain():
                    lanes = lax.iota(jnp.int32, L)
                    for v in range(fh // L):
                        lanepos = jnp.full((L,), pc, jnp.int32) + lanes + v * L
                        mask = lanepos < fl + fh
                        pos = lanepos & fmask
                        hi_i = pos >> fshift
                        lo_i = pos & (fh - 1)
                        plsc.store_scatter(
                            pend_nl, [hi_i, lo_i],
                            jnp.full((L,), rr, jnp.int32), mask=mask)
                        if rm:
                            plsc.store_scatter(
                                pend_ng, [hi_i, lo_i],
                                jnp.zeros((L,), jnp.int32), mask=mask)
                        plsc.store_scatter(
                            pend_e, [hi_i, lo_i],
                            jnp.zeros((L,), jnp.int32), mask=mask)
                    flush((fl // fh) & 1)

                plsc.subcore_barrier()

                def wb_body(bj, c2):
                    b = sid + bj * NS
                    pltpu.sync_copy(acc_sp.at[pl.ds(b * CH, CH)], wbuf)
                    pltpu.sync_copy(cnt_sp.at[pl.ds(b * CH, CH)], cbuf)

                    def row_body(r, c3):
                        cv = plsc.load_gather(
                            cbuf, [jnp.full((L,), r, jnp.int32)])
                        rv = 1.0 / jnp.maximum(cv, 1.0)
                        for g in range(8):
                            row = wbuf[r, pl.ds(g * L, L)] * rv
                            if relu:
                                row = jnp.maximum(row, 0.0)
                            wbuf[r, pl.ds(g * L, L)] = row
                        return c3

                    lax.fori_loop(0, CH, row_body, 0)
                    pltpu.sync_copy(wbuf, out_hbm.at[pl.ds(lo + b * CH, CH)])
                    return c2

                lax.fori_loop(0, (nblk_real - sid + NS - 1) // NS,
                              wb_body, 0)
                plsc.subcore_barrier()

            return qcarry

        lax.fori_loop(0, qmax, q_body, 0)

    scratch = [
        pltpu.VMEM_SHARED((racc, D), jnp.float32),
        pltpu.VMEM_SHARED((racc,), jnp.float32),
        pltpu.VMEM((CH,), jnp.int32),
        pltpu.VMEM((CH,), jnp.int32),
        pltpu.VMEM((2, fh), jnp.int32),
    ]
    if rm:
        scratch.append(pltpu.VMEM((2, fh), jnp.int32))
    scratch += [
        pltpu.VMEM((2, fh), jnp.int32),
        pltpu.VMEM((fh, D), jnp.float32),
    ]
    if rm:
        scratch += [
            pltpu.VMEM((fh, 2 * D), jnp.float32),
            pltpu.VMEM((fh, D), jnp.float32),
        ]
    scratch += [
        pltpu.VMEM((CH, D), jnp.float32),
        pltpu.VMEM((CH,), jnp.float32),
        pltpu.VMEM((fh,), jnp.float32),
        pltpu.VMEM((L,), jnp.float32),
        pltpu.SemaphoreType.DMA,
    ]
    if rm:
        scratch.append(pltpu.SemaphoreType.DMA)

    return pl.kernel(
        body,
        out_type=_f32((npad, D)),
        mesh=_mesh(),
        compiler_params=_SC_PARAMS,
        scratch_types=scratch,
    )


def _make_gather_kernel(ns_pad, ne_pad, nb):
    def body(s_hbm, e_hbm, sid_hbm, eid_hbm, out_s, out_e,
             ibuf, rows_v, sem1):
        cid = lax.axis_index("c")
        sid = lax.axis_index("s")
        base = (sid * NC + cid) * (nb // NW)
        n = nb // NW
        pltpu.sync_copy(sid_hbm.at[pl.ds(base, n)], ibuf)
        pltpu.async_copy(s_hbm.at[ibuf], rows_v, sem1).wait()
        pltpu.sync_copy(rows_v, out_s.at[pl.ds(base, n)])
        pltpu.sync_copy(eid_hbm.at[pl.ds(base, n)], ibuf)
        pltpu.async_copy(e_hbm.at[ibuf], rows_v, sem1).wait()
        pltpu.sync_copy(rows_v, out_e.at[pl.ds(base, n)])

    return pl.kernel(
        body,
        out_type=(_f32((nb, D)), _f32((nb, D))),
        mesh=_mesh(),
        compiler_params=_SC_PARAMS,
        scratch_types=[
            pltpu.VMEM((nb // NW,), jnp.int32),
            pltpu.VMEM((nb // NW, D), jnp.float32),
            pltpu.SemaphoreType.DMA,
        ],
    )


def _tc_call(body, out_shape, *args):
    return pl.pallas_call(
        body, out_shape=jax.ShapeDtypeStruct(out_shape, jnp.float32))(*args)


def _prep_g(x, w):
    def body(x_ref, w_ref, o_ref):
        o_ref[...] = (x_ref[...] @ w_ref[...]) @ w_ref[...].T

    return _tc_call(body, (K_NUM, D), x, w)


def _make_t(x, w):
    def body(x_ref, w_ref, o_ref):
        xw = x_ref[...] @ w_ref[...]
        o_ref[...] = jnp.concatenate([xw @ w_ref[...].T, xw], axis=-1)

    return _tc_call(body, (K_NUM, 2 * D), x, w)


def _post_a(mpart, cpart, w):
    def body(mp_ref, cp_ref, w_ref, o_ref):
        p = mp_ref[0] + mp_ref[1]
        c = cp_ref[0, :K_NUM] + cp_ref[1, :K_NUM]
        scale = 1.0 / jnp.maximum(c, 1.0)
        o_ref[...] = (p @ w_ref[...]) * scale[:, None]

    return _tc_call(body, (K_NUM, D), mpart, cpart, w)


def _post_rev(mpart, cpart, relu):
    def body(mp_ref, cp_ref, o_ref):
        p = mp_ref[0] + mp_ref[1]
        c = cp_ref[0, :K_NUM] + cp_ref[1, :K_NUM]
        r = p * (1.0 / jnp.maximum(c, 1.0))[:, None]
        if relu:
            r = jnp.maximum(r, 0.0)
        o_ref[...] = r

    return _tc_call(body, (K_NUM, D), mpart, cpart)


def _head(kn_r, k1, k2, stu_v, exer_v, Wfc4, bfc4, Wfc5, bfc5, Wfc3, bfc3,
          Wfc7, bfc7):
    nb = kn_r.shape[0]
    blk = 512

    def body(knr_ref, k1_ref, k2_ref, sv_ref, ev_ref, w4_ref, b4_ref,
             w5_ref, b5_ref, w3_ref, b3_ref, w7_ref, b7_ref, o_ref):
        k = 0.5 * (k1_ref[...] + k2_ref[...])
        knr = knr_ref[...]
        kn_v = (knr @ k) / (jnp.sum(knr, axis=-1, keepdims=True) + 1e-8)
        xs = jnp.tanh(jnp.concatenate([sv_ref[...], kn_v], -1) @ w4_ref[...]
                      + b4_ref[...])
        xe = jnp.tanh(jnp.concatenate([ev_ref[...], kn_v], -1) @ w5_ref[...]
                      + b5_ref[...])
        h = jax.nn.relu((xs - xe) @ w3_ref[...] + b3_ref[...])
        o_ref[...] = jax.nn.sigmoid(h @ w7_ref[...] + b7_ref[...])

    full = lambda shape: pl.BlockSpec(shape, lambda i: (0,) * len(shape))
    return pl.pallas_call(
        body,
        out_shape=jax.ShapeDtypeStruct((nb, 1), jnp.float32),
        grid=(nb // blk,),
        in_specs=[
            pl.BlockSpec((blk, K_NUM), lambda i: (i, 0)),
            full((K_NUM, D)), full((K_NUM, D)),
            pl.BlockSpec((blk, D), lambda i: (i, 0)),
            pl.BlockSpec((blk, D), lambda i: (i, 0)),
            full(Wfc4.shape), full(bfc4.shape), full(Wfc5.shape),
            full(bfc5.shape), full(Wfc3.shape), full(bfc3.shape),
            full(Wfc7.shape), full(bfc7.shape),
        ],
        out_specs=pl.BlockSpec((blk, 1), lambda i: (i, 0)),
    )(kn_r, k1, k2, stu_v, exer_v, Wfc4, bfc4, Wfc5, bfc5, Wfc3, bfc3,
      Wfc7, bfc7)


RC_F2 = 8192
RC_RM = 5632


def _side(x_table, kn_table, nidx, eidx, nnz, npad_f2, npad_rm, W1, W2,
          Wr1, Wr2, z2d, z1d):
    nx = x_table.shape[0]
    a_x = _make_a_kernel(nnz, nx, alpha=True)
    a_s = _make_a_kernel(nnz, npad_f2, alpha=True)
    a_m = _make_a_kernel(nnz, npad_rm, alpha=False)
    b_f2 = _make_b_kernel(nnz, npad_f2, RC_F2, "f2", relu=False)
    b_f2r = _make_b_kernel(nnz, npad_f2, RC_F2, "f2", relu=True)
    b_rm = _make_b_kernel(nnz, npad_rm, RC_RM, "rm", relu=False)

    hg1 = _prep_g(kn_table, W1)
    mp, cp = a_x(x_table, hg1, nidx, eidx, z2d, z1d)
    m1 = _post_a(mp, cp, W1)
    s1 = b_f2(m1, nidx, eidx, z2d, z1d)

    hg2 = _prep_g(kn_table, W2)
    mp, cp = a_s(s1, hg2, nidx, eidx, z2d, z1d)
    m2 = _post_a(mp, cp, W2)
    s = b_f2r(m2, nidx, eidx, z2d, z1d)

    t3 = _make_t(kn_table, Wr1)
    m3 = b_rm(s, t3, nidx, eidx, z2d, z1d)
    mp, cp = a_m(m3, nidx, eidx, z2d, z1d)
    k1a = _post_rev(mp, cp, relu=False)

    t4 = _make_t(k1a, Wr2)
    m4 = b_rm(s, t4, nidx, eidx, z2d, z1d)
    mp, cp = a_m(m4, nidx, eidx, z2d, z1d)
    k1 = _post_rev(mp, cp, relu=True)
    return s, k1


def kernel(stu_id, exer_id, kn_r, hidx_sk, hidx_ek, stu_table, exer_table,
           kn_table, Ws1, Ws2, Wks1, Wks2, We1, We2, Wke1, Wke2,
           Wfc4, bfc4, Wfc5, bfc5, Wfc3, bfc3, Wfc7, bfc7):
    z2d = jnp.zeros((CH, D), jnp.float32)
    z1d = jnp.zeros((CH,), jnp.float32)

    n_sk = hidx_sk[0]
    e_sk = hidx_sk[1]
    n_ek = hidx_ek[0]
    e_ek = hidx_ek[1]

    s, k1 = _side(stu_table, kn_table, n_sk, e_sk, 400000, 57344, 50688,
                  Ws1, Ws2, Wks1, Wks2, z2d, z1d)
    e, k2 = _side(exer_table, kn_table, n_ek, e_ek, 160000, 24576, 22528,
                  We1, We2, Wke1, Wke2, z2d, z1d)

    nb = stu_id.shape[0]
    gk = _make_gather_kernel(57344, 24576, nb)
    stu_v, exer_v = gk(s, e, stu_id.astype(jnp.int32),
                       exer_id.astype(jnp.int32))

    return _head(kn_r, k1, k2, stu_v, exer_v, Wfc4, bfc4, Wfc5, bfc5,
                 Wfc3, bfc3, Wfc7, bfc7)


# speedup vs baseline: 2.1182x; 1.3395x over previous
import functools
import math

import jax
import jax.numpy as jnp
from jax import lax
from jax.experimental import pallas as pl
from jax.experimental.pallas import tpu as pltpu
from jax.experimental.pallas import tpu_sc as plsc

NC = 2
NS = 16
NW = NC * NS
L = 16
D = 128
CH = 128
K_NUM = 2000
KPAD = 2048
RSQ = 1.0 / math.sqrt(D)

_mesh = functools.partial(
    plsc.VectorSubcoreMesh, core_axis_name="c", subcore_axis_name="s",
    num_cores=NC, num_subcores=NS)
_SC_PARAMS = pltpu.CompilerParams(needs_layout_passes=False)


def _f32(shape):
    return jax.ShapeDtypeStruct(shape, jnp.float32)


def _fill_ones(ref):
    for i in range(ref.shape[0] // L):
        ref[pl.ds(i * L, L)] = jnp.full((L,), 1.0, jnp.float32)


def _hsum_splat(tmp_ref, v):
    lanes = lax.iota(jnp.int32, L)
    for sh in (8, 4, 2, 1):
        tmp_ref[...] = v
        v = v + plsc.load_gather(tmp_ref, [lanes ^ sh])
    return v


def _split2k(sid, copy_fn):
    @pl.when(sid < 15)
    def _():
        copy_fn(sid * CH, CH)

    @pl.when(sid == 15)
    def _():
        copy_fn(15 * CH, 80)


def _make_a_kernel(nnz, nx, alpha):
    def body(*refs):
        if alpha:
            (x_hbm, g_hbm, pidx, z2d, z1d, mpart, cpart,
             m_sp, cnt_sp, pbuf, nbuf, ebuf, xbuf, gbuf, abuf, ones_v,
             tmp16, sem1, sem2) = refs
        else:
            (x_hbm, pidx, z2d, z1d, mpart, cpart,
             m_sp, cnt_sp, pbuf, nbuf, ebuf, xbuf, ones_v, sem1) = refs
        cid = lax.axis_index("c")
        sid = lax.axis_index("s")
        w = sid * NC + cid
        def _zm(off, sz):
            pltpu.sync_copy(z2d.at[pl.ds(0, sz)], m_sp.at[pl.ds(off, sz)])

        _split2k(sid, _zm)
        pltpu.sync_copy(z1d, cnt_sp.at[pl.ds(sid * CH, CH)])
        _fill_ones(ones_v)
        plsc.subcore_barrier()

        nchunks = nnz // CH
        trips = (nchunks - w + NW - 1) // NW

        def chunk_body(j, carry):
            base = (w + j * NW) * CH
            pltpu.sync_copy(pidx.at[pl.ds(base, CH)], pbuf)
            for v in range(8):
                pv = pbuf[pl.ds(v * L, L)]
                nbuf[pl.ds(v * L, L)] = pv >> 11
                ebuf[pl.ds(v * L, L)] = pv & 2047
            cpx = pltpu.async_copy(x_hbm.at[nbuf], xbuf, sem1)
            if alpha:
                cpg = pltpu.async_copy(g_hbm.at[ebuf], gbuf, sem2)
            cpx.wait()
            if alpha:
                cpg.wait()

                def row_body(r, c2):
                    acc = jnp.zeros((L,), jnp.float32)
                    xr = []
                    for g in range(8):
                        xv = xbuf[r, pl.ds(g * L, L)]
                        xr.append(xv)
                        acc = acc + xv * gbuf[r, pl.ds(g * L, L)]
                    z = _hsum_splat(tmp16, acc)
                    av = 1.0 / (1.0 + jnp.exp(-z * RSQ))
                    for g in range(8):
                        abuf[r, pl.ds(g * L, L)] = xr[g] * av
                    return c2

                lax.fori_loop(0, CH, row_body, 0)
                src = abuf
            else:
                src = xbuf
            pltpu.sync_copy(src, m_sp.at[ebuf], add=True)
            pltpu.sync_copy(ones_v, cnt_sp.at[ebuf], add=True)
            return carry

        lax.fori_loop(0, trips, chunk_body, 0)
        plsc.subcore_barrier()

        def _wm(off, sz):
            pltpu.sync_copy(m_sp.at[pl.ds(off, sz)],
                            mpart.at[cid, pl.ds(off, sz)])

        _split2k(sid, _wm)
        pltpu.sync_copy(cnt_sp.at[pl.ds(sid * CH, CH)],
                        cpart.at[cid, pl.ds(sid * CH, CH)])

    scratch = [
        pltpu.VMEM_SHARED((K_NUM, D), jnp.float32),
    ]
    scratch += [
        pltpu.VMEM_SHARED((KPAD,), jnp.float32),
        pltpu.VMEM((CH,), jnp.int32),
        pltpu.VMEM((CH,), jnp.int32),
        pltpu.VMEM((CH,), jnp.int32),
        pltpu.VMEM((CH, D), jnp.float32),
    ]
    if alpha:
        scratch += [
            pltpu.VMEM((CH, D), jnp.float32),
            pltpu.VMEM((CH, D), jnp.float32),
        ]
    scratch += [
        pltpu.VMEM((CH,), jnp.float32),
    ]
    if alpha:
        scratch.append(pltpu.VMEM((L,), jnp.float32))
    scratch.append(pltpu.SemaphoreType.DMA)
    if alpha:
        scratch.append(pltpu.SemaphoreType.DMA)

    return pl.kernel(
        body,
        out_type=(_f32((NC, K_NUM, D)), _f32((NC, KPAD))),
        mesh=_mesh(),
        compiler_params=_SC_PARAMS,
        scratch_types=scratch,
    )


def _make_b_kernel(nnz, npad, rchunk, mode, relu, fh_f2=128):
    rr = rchunk
    racc = rr + CH
    nch = npad // rr
    qmax = (nch + NC - 1) // NC
    rm = mode == "rm"
    fh = 64 if rm else fh_f2
    wb = 64 if rm else 128
    fmask = 2 * fh - 1
    fshift = fh.bit_length() - 1

    def body(*refs):
        if rm:
            (s_hbm, t_hbm, pidx, z2d, z1d, out_hbm,
             acc_sp, cnt_sp, pbuf, pend_nl, pend_ng, pend_e,
             rows_v, trows_v, abuf, wbuf, cbuf, ones_v, tmp16,
             sem1, sem2) = refs
        else:
            (m_hbm, pidx, z2d, z1d, out_hbm,
             acc_sp, cnt_sp, pbuf, pend_nl, pend_e,
             rows_v, wbuf, cbuf, ones_v, tmp16, sem1) = refs
        cid = lax.axis_index("c")
        sid = lax.axis_index("s")
        small_hbm = t_hbm if rm else m_hbm
        _fill_ones(ones_v)

        def flush(par):
            idx_e = pend_e.at[par]
            idx_nl = pend_nl.at[par]
            if rm:
                cp1 = pltpu.async_copy(s_hbm.at[pend_ng.at[par]], rows_v,
                                       sem1)
                cp2 = pltpu.async_copy(t_hbm.at[idx_e], trows_v, sem2)
                cp1.wait()
                cp2.wait()

                def row_body(r, c2):
                    acc = jnp.zeros((L,), jnp.float32)
                    tw = []
                    for g in range(8):
                        acc = acc + (rows_v[r, pl.ds(g * L, L)] *
                                     trows_v[r, pl.ds(g * L, L)])
                        tw.append(trows_v[r, pl.ds(D + g * L, L)])
                    z = _hsum_splat(tmp16, acc)
                    av = 1.0 / (1.0 + jnp.exp(-z * RSQ))
                    for g in range(8):
                        abuf[r, pl.ds(g * L, L)] = tw[g] * av
                    return c2

                lax.fori_loop(0, fh, row_body, 0)
                pltpu.sync_copy(abuf, acc_sp.at[idx_nl], add=True)
            else:
                cp = pltpu.async_copy(m_hbm.at[idx_e], rows_v, sem1)
                cp.wait()
                pltpu.sync_copy(rows_v, acc_sp.at[idx_nl], add=True)
            pltpu.sync_copy(ones_v, cnt_sp.at[idx_nl], add=True)

        nblk_acc = racc // CH
        nblk_real = rr // wb
        scan_chunks = nnz // CH

        def q_body(q, qcarry):
            ch = cid + NC * q

            @pl.when(ch < nch)
            def _do_chunk():
                lo = ch * rr
                hi = lo + rr

                def zero_body(bj, c2):
                    b = sid + bj * NS
                    pltpu.sync_copy(z2d, acc_sp.at[pl.ds(b * CH, CH)])
                    pltpu.sync_copy(z1d, cnt_sp.at[pl.ds(b * CH, CH)])
                    return c2

                lax.fori_loop(0, (nblk_acc - sid + NS - 1) // NS,
                              zero_body, 0)
                plsc.subcore_barrier()

                def scan_body(j, carry):
                    pc, fl = carry
                    base = (sid + j * NS) * CH
                    pltpu.sync_copy(pidx.at[pl.ds(base, CH)], pbuf)
                    for v in range(8):
                        pv = pbuf[pl.ds(v * L, L)]
                        nv = pv >> 11
                        ev = pv & 2047
                        within = (nv >= lo) & (nv < hi)
                        wi = within.astype(jnp.int32)
                        cs = plsc.cumsum(wi)
                        pos = (jnp.full((L,), pc, jnp.int32) + cs - 1) & fmask
                        hi_i = pos >> fshift
                        lo_i = pos & (fh - 1)
                        plsc.store_scatter(pend_nl, [hi_i, lo_i], nv - lo,
                                           mask=within)
                        if rm:
                            plsc.store_scatter(pend_ng, [hi_i, lo_i], nv,
                                               mask=within)
                        plsc.store_scatter(pend_e, [hi_i, lo_i], ev,
                                           mask=within)
                        pc = pc + cs[L - 1]
                        do = (pc - fl) >= fh

                        @pl.when(do)
                        def _():
                            flush((fl // fh) & 1)

                        fl = jnp.where(do, fl + fh, fl)
                    return pc, fl

                pc, fl = lax.fori_loop(
                    0, (scan_chunks - sid + NS - 1) // NS, scan_body,
                    (jnp.int32(0), jnp.int32(0)))

                @pl.when(pc > fl)
                def _drain():
                    lanes = lax.iota(jnp.int32, L)
                    for v in range(fh // L):
                        lanepos = jnp.full((L,), pc, jnp.int32) + lanes + v * L
                        mask = lanepos < fl + fh
                        pos = lanepos & fmask
                        hi_i = pos >> fshift
                        lo_i = pos & (fh - 1)
                        plsc.store_scatter(
                            pend_nl, [hi_i, lo_i],
                            jnp.full((L,), rr, jnp.int32), mask=mask)
                        if rm:
                            plsc.store_scatter(
                                pend_ng, [hi_i, lo_i],
                                jnp.zeros((L,), jnp.int32), mask=mask)
                        plsc.store_scatter(
                            pend_e, [hi_i, lo_i],
                            jnp.zeros((L,), jnp.int32), mask=mask)
                    flush((fl // fh) & 1)

                plsc.subcore_barrier()

                def wb_body(bj, c2):
                    b = sid + bj * NS
                    pltpu.sync_copy(acc_sp.at[pl.ds(b * wb, wb)], wbuf)
                    pltpu.sync_copy(cnt_sp.at[pl.ds(b * wb, wb)], cbuf)

                    def row_body(r, c3):
                        cv = plsc.load_gather(
                            cbuf, [jnp.full((L,), r, jnp.int32)])
                        rv = 1.0 / jnp.maximum(cv, 1.0)
                        for g in range(8):
                            row = wbuf[r, pl.ds(g * L, L)] * rv
                            if relu:
                                row = jnp.maximum(row, 0.0)
                            wbuf[r, pl.ds(g * L, L)] = row
                        return c3

                    lax.fori_loop(0, wb, row_body, 0)
                    pltpu.sync_copy(wbuf, out_hbm.at[pl.ds(lo + b * wb, wb)])
                    return c2

                lax.fori_loop(0, (nblk_real - sid + NS - 1) // NS,
                              wb_body, 0)
                plsc.subcore_barrier()

            return qcarry

        lax.fori_loop(0, qmax, q_body, 0)

    scratch = [
        pltpu.VMEM_SHARED((racc, D), jnp.float32),
        pltpu.VMEM_SHARED((racc,), jnp.float32),
        pltpu.VMEM((CH,), jnp.int32),
        pltpu.VMEM((2, fh), jnp.int32),
    ]
    if rm:
        scratch.append(pltpu.VMEM((2, fh), jnp.int32))
    scratch += [
        pltpu.VMEM((2, fh), jnp.int32),
        pltpu.VMEM((fh, D), jnp.float32),
    ]
    if rm:
        scratch += [
            pltpu.VMEM((fh, 2 * D), jnp.float32),
            pltpu.VMEM((fh, D), jnp.float32),
        ]
    scratch += [
        pltpu.VMEM((wb, D), jnp.float32),
        pltpu.VMEM((wb,), jnp.float32),
        pltpu.VMEM((fh,), jnp.float32),
        pltpu.VMEM((L,), jnp.float32),
        pltpu.SemaphoreType.DMA,
    ]
    if rm:
        scratch.append(pltpu.SemaphoreType.DMA)

    return pl.kernel(
        body,
        out_type=_f32((npad, D)),
        mesh=_mesh(),
        compiler_params=_SC_PARAMS,
        scratch_types=scratch,
    )


def _make_gather_kernel(ns_pad, ne_pad, nb):
    def body(s_hbm, e_hbm, sid_hbm, eid_hbm, out_s, out_e,
             ibuf, rows_v, sem1):
        cid = lax.axis_index("c")
        sid = lax.axis_index("s")
        base = (sid * NC + cid) * (nb // NW)
        n = nb // NW
        pltpu.sync_copy(sid_hbm.at[pl.ds(base, n)], ibuf)
        pltpu.async_copy(s_hbm.at[ibuf], rows_v, sem1).wait()
        pltpu.sync_copy(rows_v, out_s.at[pl.ds(base, n)])
        pltpu.sync_copy(eid_hbm.at[pl.ds(base, n)], ibuf)
        pltpu.async_copy(e_hbm.at[ibuf], rows_v, sem1).wait()
        pltpu.sync_copy(rows_v, out_e.at[pl.ds(base, n)])

    return pl.kernel(
        body,
        out_type=(_f32((nb, D)), _f32((nb, D))),
        mesh=_mesh(),
        compiler_params=_SC_PARAMS,
        scratch_types=[
            pltpu.VMEM((nb // NW,), jnp.int32),
            pltpu.VMEM((nb // NW, D), jnp.float32),
            pltpu.SemaphoreType.DMA,
        ],
    )


def _tc_call(body, out_shape, *args):
    return pl.pallas_call(
        body, out_shape=jax.ShapeDtypeStruct(out_shape, jnp.float32))(*args)


def _prep_g(x, w):
    def body(x_ref, w_ref, o_ref):
        o_ref[...] = (x_ref[...] @ w_ref[...]) @ w_ref[...].T

    return _tc_call(body, (K_NUM, D), x, w)


def _make_t(x, w):
    def body(x_ref, w_ref, o_ref):
        xw = x_ref[...] @ w_ref[...]
        o_ref[...] = jnp.concatenate([xw @ w_ref[...].T, xw], axis=-1)

    return _tc_call(body, (K_NUM, 2 * D), x, w)


def _post_a(mpart, cpart, w):
    def body(mp_ref, cp_ref, w_ref, o_ref):
        p = mp_ref[0] + mp_ref[1]
        c = cp_ref[0, :K_NUM] + cp_ref[1, :K_NUM]
        scale = 1.0 / jnp.maximum(c, 1.0)
        o_ref[...] = (p @ w_ref[...]) * scale[:, None]

    return _tc_call(body, (K_NUM, D), mpart, cpart, w)


def _post_rev(mpart, cpart, relu):
    def body(mp_ref, cp_ref, o_ref):
        p = mp_ref[0] + mp_ref[1]
        c = cp_ref[0, :K_NUM] + cp_ref[1, :K_NUM]
        r = p * (1.0 / jnp.maximum(c, 1.0))[:, None]
        if relu:
            r = jnp.maximum(r, 0.0)
        o_ref[...] = r

    return _tc_call(body, (K_NUM, D), mpart, cpart)


def _head(kn_r, k1, k2, stu_v, exer_v, Wfc4, bfc4, Wfc5, bfc5, Wfc3, bfc3,
          Wfc7, bfc7):
    nb = kn_r.shape[0]
    blk = 512

    def body(knr_ref, k1_ref, k2_ref, sv_ref, ev_ref, w4_ref, b4_ref,
             w5_ref, b5_ref, w3_ref, b3_ref, w7_ref, b7_ref, o_ref):
        k = 0.5 * (k1_ref[...] + k2_ref[...])
        knr = knr_ref[...]
        kn_v = (knr @ k) / (jnp.sum(knr, axis=-1, keepdims=True) + 1e-8)
        xs = jnp.tanh(jnp.concatenate([sv_ref[...], kn_v], -1) @ w4_ref[...]
                      + b4_ref[...])
        xe = jnp.tanh(jnp.concatenate([ev_ref[...], kn_v], -1) @ w5_ref[...]
                      + b5_ref[...])
        h = jax.nn.relu((xs - xe) @ w3_ref[...] + b3_ref[...])
        o_ref[...] = jax.nn.sigmoid(h @ w7_ref[...] + b7_ref[...])

    full = lambda shape: pl.BlockSpec(shape, lambda i: (0,) * len(shape))
    return pl.pallas_call(
        body,
        out_shape=jax.ShapeDtypeStruct((nb, 1), jnp.float32),
        grid=(nb // blk,),
        in_specs=[
            pl.BlockSpec((blk, K_NUM), lambda i: (i, 0)),
            full((K_NUM, D)), full((K_NUM, D)),
            pl.BlockSpec((blk, D), lambda i: (i, 0)),
            pl.BlockSpec((blk, D), lambda i: (i, 0)),
            full(Wfc4.shape), full(bfc4.shape), full(Wfc5.shape),
            full(bfc5.shape), full(Wfc3.shape), full(bfc3.shape),
            full(Wfc7.shape), full(bfc7.shape),
        ],
        out_specs=pl.BlockSpec((blk, 1), lambda i: (i, 0)),
    )(kn_r, k1, k2, stu_v, exer_v, Wfc4, bfc4, Wfc5, bfc5, Wfc3, bfc3,
      Wfc7, bfc7)


def _side(x_table, kn_table, pidx, nnz, npad, rchunk, W1, W2,
          Wr1, Wr2, z2d, z1d, fh_f2=128):
    nx = x_table.shape[0]
    a_x = _make_a_kernel(nnz, nx, alpha=True)
    a_s = _make_a_kernel(nnz, npad, alpha=True)
    a_m = _make_a_kernel(nnz, npad, alpha=False)
    b_f2 = _make_b_kernel(nnz, npad, rchunk, "f2", relu=False, fh_f2=fh_f2)
    b_f2r = _make_b_kernel(nnz, npad, rchunk, "f2", relu=True, fh_f2=fh_f2)
    b_rm = _make_b_kernel(nnz, npad, rchunk, "rm", relu=False)

    hg1 = _prep_g(kn_table, W1)
    mp, cp = a_x(x_table, hg1, pidx, z2d, z1d)
    m1 = _post_a(mp, cp, W1)
    s1 = b_f2(m1, pidx, z2d, z1d)

    hg2 = _prep_g(kn_table, W2)
    mp, cp = a_s(s1, hg2, pidx, z2d, z1d)
    m2 = _post_a(mp, cp, W2)
    s = b_f2r(m2, pidx, z2d, z1d)

    t3 = _make_t(kn_table, Wr1)
    m3 = b_rm(s, t3, pidx, z2d, z1d)
    mp, cp = a_m(m3, pidx, z2d, z1d)
    k1a = _post_rev(mp, cp, relu=False)

    t4 = _make_t(k1a, Wr2)
    m4 = b_rm(s, t4, pidx, z2d, z1d)
    mp, cp = a_m(m4, pidx, z2d, z1d)
    k1 = _post_rev(mp, cp, relu=True)
    return s, k1


def kernel(stu_id, exer_id, kn_r, hidx_sk, hidx_ek, stu_table, exer_table,
           kn_table, Ws1, Ws2, Wks1, Wks2, We1, We2, Wke1, Wke2,
           Wfc4, bfc4, Wfc5, bfc5, Wfc3, bfc3, Wfc7, bfc7):
    z2d = jnp.zeros((CH, D), jnp.float32)
    z1d = jnp.zeros((CH,), jnp.float32)

    p_sk = (hidx_sk[0].astype(jnp.int32) * 2048 +
            hidx_sk[1].astype(jnp.int32))
    p_ek = (hidx_ek[0].astype(jnp.int32) * 2048 +
            hidx_ek[1].astype(jnp.int32))

    s, k1 = _side(stu_table, kn_table, p_sk, 400000, 58368, 9728,
                  Ws1, Ws2, Wks1, Wks2, z2d, z1d)
    e, k2 = _side(exer_table, kn_table, p_ek, 160000, 20224, 10112,
                  We1, We2, Wke1, Wke2, z2d, z1d, fh_f2=64)

    nb = stu_id.shape[0]
    gk = _make_gather_kernel(58368, 20224, nb)
    stu_v, exer_v = gk(s, e, stu_id.astype(jnp.int32),
                       exer_id.astype(jnp.int32))

    return _head(kn_r, k1, k2, stu_v, exer_v, Wfc4, bfc4, Wfc5, bfc5,
                 Wfc3, bfc3, Wfc7, bfc7)


# speedup vs baseline: 2.2013x; 1.0392x over previous
import functools
import math

import jax
import jax.numpy as jnp
from jax import lax
from jax.experimental import pallas as pl
from jax.experimental.pallas import tpu as pltpu
from jax.experimental.pallas import tpu_sc as plsc

NC = 2
NS = 16
NW = NC * NS
L = 16
D = 128
CH = 128
K_NUM = 2000
KPAD = 2048
RSQ = 1.0 / math.sqrt(D)

_mesh = functools.partial(
    plsc.VectorSubcoreMesh, core_axis_name="c", subcore_axis_name="s",
    num_cores=NC, num_subcores=NS)
_SC_PARAMS = pltpu.CompilerParams(needs_layout_passes=False)


def _f32(shape):
    return jax.ShapeDtypeStruct(shape, jnp.float32)


def _fill_ones(ref):
    for i in range(ref.shape[0] // L):
        ref[pl.ds(i * L, L)] = jnp.full((L,), 1.0, jnp.float32)


def _hsum_splat(tmp_ref, v):
    lanes = lax.iota(jnp.int32, L)
    for sh in (8, 4, 2, 1):
        tmp_ref[...] = v
        v = v + plsc.load_gather(tmp_ref, [lanes ^ sh])
    return v


def _split2k(sid, copy_fn):
    @pl.when(sid < 15)
    def _():
        copy_fn(sid * CH, CH)

    @pl.when(sid == 15)
    def _():
        copy_fn(15 * CH, 80)


def _make_a_kernel(nnz, nx, alpha):
    def body(*refs):
        if alpha:
            (x_hbm, g_hbm, pidx, z2d, z1d, mpart, cpart,
             m_sp, cnt_sp, pbuf, nbuf0, ebuf0, xbuf0, gbuf0,
             nbuf1, ebuf1, xbuf1, gbuf1, abuf, ones_v,
             tmp16, tmp16b, semx0, semx1, semg0, semg1) = refs
            nbufs, ebufs = (nbuf0, nbuf1), (ebuf0, ebuf1)
            xbufs, gbufs = (xbuf0, xbuf1), (gbuf0, gbuf1)
            semxs, semgs = (semx0, semx1), (semg0, semg1)
        else:
            (x_hbm, pidx, z2d, z1d, mpart, cpart,
             m_sp, cnt_sp, pbuf, nbuf0, ebuf0, xbuf0,
             nbuf1, ebuf1, xbuf1, ones_v, semx0, semx1) = refs
            nbufs, ebufs = (nbuf0, nbuf1), (ebuf0, ebuf1)
            xbufs = (xbuf0, xbuf1)
            semxs = (semx0, semx1)
        cid = lax.axis_index("c")
        sid = lax.axis_index("s")
        w = sid * NC + cid

        def _zm(off, sz):
            pltpu.sync_copy(z2d.at[pl.ds(0, sz)], m_sp.at[pl.ds(off, sz)])

        _split2k(sid, _zm)
        pltpu.sync_copy(z1d, cnt_sp.at[pl.ds(sid * CH, CH)])
        _fill_ones(ones_v)
        plsc.subcore_barrier()

        nchunks = nnz // CH
        trips = (nchunks - w + NW - 1) // NW

        def issue(j, slot):
            base = (w + j * NW) * CH
            pltpu.sync_copy(pidx.at[pl.ds(base, CH)], pbuf)
            for v in range(8):
                pv = pbuf[pl.ds(v * L, L)]
                nbufs[slot][pl.ds(v * L, L)] = pv >> 11
                ebufs[slot][pl.ds(v * L, L)] = pv & 2047
            pltpu.async_copy(x_hbm.at[nbufs[slot]], xbufs[slot], semxs[slot])
            if alpha:
                pltpu.async_copy(g_hbm.at[ebufs[slot]], gbufs[slot],
                                 semgs[slot])

        def process(slot):
            xb = xbufs[slot]
            pltpu.make_async_copy(x_hbm.at[nbufs[slot]], xb,
                                  semxs[slot]).wait()
            if alpha:
                gb = gbufs[slot]
                pltpu.make_async_copy(g_hbm.at[ebufs[slot]], gb,
                                      semgs[slot]).wait()

                def row_body(r2, c2):
                    for half, tref in ((0, tmp16), (1, tmp16b)):
                        r = r2 * 2 + half
                        acc = jnp.zeros((L,), jnp.float32)
                        xr = []
                        for g in range(8):
                            xv = xb[r, pl.ds(g * L, L)]
                            xr.append(xv)
                            acc = acc + xv * gb[r, pl.ds(g * L, L)]
                        z = _hsum_splat(tref, acc)
                        av = 1.0 / (1.0 + jnp.exp(-z * RSQ))
                        for g in range(8):
                            abuf[r, pl.ds(g * L, L)] = xr[g] * av
                    return c2

                lax.fori_loop(0, CH // 2, row_body, 0)
                src_buf = abuf
            else:
                src_buf = xb
            pltpu.sync_copy(src_buf, m_sp.at[ebufs[slot]], add=True)
            pltpu.sync_copy(ones_v, cnt_sp.at[ebufs[slot]], add=True)

        @pl.when(trips > 0)
        def _prologue():
            issue(0, 0)

        def pair_body(j2, carry):
            j0 = j2 * 2

            @pl.when(j0 + 1 < trips)
            def _():
                issue(j0 + 1, 1)

            process(0)

            @pl.when(j0 + 1 < trips)
            def _():
                @pl.when(j0 + 2 < trips)
                def _():
                    issue(j0 + 2, 0)

                process(1)

            return carry

        lax.fori_loop(0, (trips + 1) // 2, pair_body, 0)
        plsc.subcore_barrier()

        def _wm(off, sz):
            pltpu.sync_copy(m_sp.at[pl.ds(off, sz)],
                            mpart.at[cid, pl.ds(off, sz)])

        _split2k(sid, _wm)
        pltpu.sync_copy(cnt_sp.at[pl.ds(sid * CH, CH)],
                        cpart.at[cid, pl.ds(sid * CH, CH)])

    scratch = [
        pltpu.VMEM_SHARED((K_NUM, D), jnp.float32),
        pltpu.VMEM_SHARED((KPAD,), jnp.float32),
        pltpu.VMEM((CH,), jnp.int32),
    ]
    if alpha:
        scratch += [
            pltpu.VMEM((CH,), jnp.int32),
            pltpu.VMEM((CH,), jnp.int32),
            pltpu.VMEM((CH, D), jnp.float32),
            pltpu.VMEM((CH, D), jnp.float32),
            pltpu.VMEM((CH,), jnp.int32),
            pltpu.VMEM((CH,), jnp.int32),
            pltpu.VMEM((CH, D), jnp.float32),
            pltpu.VMEM((CH, D), jnp.float32),
            pltpu.VMEM((CH, D), jnp.float32),
            pltpu.VMEM((CH,), jnp.float32),
            pltpu.VMEM((L,), jnp.float32),
            pltpu.VMEM((L,), jnp.float32),
            pltpu.SemaphoreType.DMA,
            pltpu.SemaphoreType.DMA,
            pltpu.SemaphoreType.DMA,
            pltpu.SemaphoreType.DMA,
        ]
    else:
        scratch += [
            pltpu.VMEM((CH,), jnp.int32),
            pltpu.VMEM((CH,), jnp.int32),
            pltpu.VMEM((CH, D), jnp.float32),
            pltpu.VMEM((CH,), jnp.int32),
            pltpu.VMEM((CH,), jnp.int32),
            pltpu.VMEM((CH, D), jnp.float32),
            pltpu.VMEM((CH,), jnp.float32),
            pltpu.SemaphoreType.DMA,
            pltpu.SemaphoreType.DMA,
        ]

    return pl.kernel(
        body,
        out_type=(_f32((NC, K_NUM, D)), _f32((NC, KPAD))),
        mesh=_mesh(),
        compiler_params=_SC_PARAMS,
        scratch_types=scratch,
    )


def _make_b_kernel(nnz, npad, rchunk, mode, relu, fh_f2=128):
    rr = rchunk
    racc = rr + CH
    nch = npad // rr
    qmax = (nch + NC - 1) // NC
    rm = mode == "rm"
    fh = 64 if rm else fh_f2
    wb = 64 if rm else 128
    fmask = 2 * fh - 1
    fshift = fh.bit_length() - 1

    def body(*refs):
        if rm:
            (s_hbm, t_hbm, pidx, z2d, z1d, out_hbm,
             acc_sp, cnt_sp, pbuf, pend_nl, pend_ng, pend_e,
             rows_v, trows_v, abuf, wbuf, cbuf, ones_v, tmp16, tmp16b,
             sem1, sem2) = refs
        else:
            (m_hbm, pidx, z2d, z1d, out_hbm,
             acc_sp, cnt_sp, pbuf, pend_nl, pend_e,
             rows_v, wbuf, cbuf, ones_v, tmp16, sem1) = refs
        cid = lax.axis_index("c")
        sid = lax.axis_index("s")
        small_hbm = t_hbm if rm else m_hbm
        _fill_ones(ones_v)

        def flush(par):
            idx_e = pend_e.at[par]
            idx_nl = pend_nl.at[par]
            if rm:
                cp1 = pltpu.async_copy(s_hbm.at[pend_ng.at[par]], rows_v,
                                       sem1)
                cp2 = pltpu.async_copy(t_hbm.at[idx_e], trows_v, sem2)
                cp1.wait()
                cp2.wait()

                def row_body(r2, c2):
                    for half, tref in ((0, tmp16), (1, tmp16b)):
                        r = r2 * 2 + half
                        acc = jnp.zeros((L,), jnp.float32)
                        tw = []
                        for g in range(8):
                            acc = acc + (rows_v[r, pl.ds(g * L, L)] *
                                         trows_v[r, pl.ds(g * L, L)])
                            tw.append(trows_v[r, pl.ds(D + g * L, L)])
                        z = _hsum_splat(tref, acc)
                        av = 1.0 / (1.0 + jnp.exp(-z * RSQ))
                        for g in range(8):
                            abuf[r, pl.ds(g * L, L)] = tw[g] * av
                    return c2

                lax.fori_loop(0, fh // 2, row_body, 0)
                pltpu.sync_copy(abuf, acc_sp.at[idx_nl], add=True)
            else:
                cp = pltpu.async_copy(m_hbm.at[idx_e], rows_v, sem1)
                cp.wait()
                pltpu.sync_copy(rows_v, acc_sp.at[idx_nl], add=True)
            pltpu.sync_copy(ones_v, cnt_sp.at[idx_nl], add=True)

        nblk_acc = racc // CH
        nblk_real = rr // wb
        scan_chunks = nnz // CH

        def q_body(q, qcarry):
            ch = cid + NC * q

            @pl.when(ch < nch)
            def _do_chunk():
                lo = ch * rr
                hi = lo + rr

                def zero_body(bj, c2):
                    b = sid + bj * NS
                    pltpu.sync_copy(z2d, acc_sp.at[pl.ds(b * CH, CH)])
                    pltpu.sync_copy(z1d, cnt_sp.at[pl.ds(b * CH, CH)])
                    return c2

                lax.fori_loop(0, (nblk_acc - sid + NS - 1) // NS,
                              zero_body, 0)
                plsc.subcore_barrier()

                def scan_body(j, carry):
                    pc, fl = carry
                    base = (sid + j * NS) * CH
                    pltpu.sync_copy(pidx.at[pl.ds(base, CH)], pbuf)
                    for v in range(8):
                        pv = pbuf[pl.ds(v * L, L)]
                        nv = pv >> 11
                        ev = pv & 2047
                        within = (nv >= lo) & (nv < hi)
                        wi = within.astype(jnp.int32)
                        cs = plsc.cumsum(wi)
                        pos = (jnp.full((L,), pc, jnp.int32) + cs - 1) & fmask
                        hi_i = pos >> fshift
                        lo_i = pos & (fh - 1)
                        plsc.store_scatter(pend_nl, [hi_i, lo_i], nv - lo,
                                           mask=within)
                        if rm:
                            plsc.store_scatter(pend_ng, [hi_i, lo_i], nv,
                                               mask=within)
                        plsc.store_scatter(pend_e, [hi_i, lo_i], ev,
                                           mask=within)
                        pc = pc + cs[L - 1]
                        do = (pc - fl) >= fh

                        @pl.when(do)
                        def _():
                            flush((fl // fh) & 1)

                        fl = jnp.where(do, fl + fh, fl)
                    return pc, fl

                pc, fl = lax.fori_loop(
                    0, (scan_chunks - sid + NS - 1) // NS, scan_body,
                    (jnp.int32(0), jnp.int32(0)))

                @pl.when(pc > fl)
                def _drain():
                    lanes = lax.iota(jnp.int32, L)
                    for v in range(fh // L):
                        lanepos = jnp.full((L,), pc, jnp.int32) + lanes + v * L
                        mask = lanepos < fl + fh
                        pos = lanepos & fmask
                        hi_i = pos >> fshift
                        lo_i = pos & (fh - 1)
                        plsc.store_scatter(
                            pend_nl, [hi_i, lo_i],
                            jnp.full((L,), rr, jnp.int32), mask=mask)
                        if rm:
                            plsc.store_scatter(
                                pend_ng, [hi_i, lo_i],
                                jnp.zeros((L,), jnp.int32), mask=mask)
                        plsc.store_scatter(
                            pend_e, [hi_i, lo_i],
                            jnp.zeros((L,), jnp.int32), mask=mask)
                    flush((fl // fh) & 1)

                plsc.subcore_barrier()

                def wb_body(bj, c2):
                    b = sid + bj * NS
                    pltpu.sync_copy(acc_sp.at[pl.ds(b * wb, wb)], wbuf)
                    pltpu.sync_copy(cnt_sp.at[pl.ds(b * wb, wb)], cbuf)

                    def row_body(r, c3):
                        cv = plsc.load_gather(
                            cbuf, [jnp.full((L,), r, jnp.int32)])
                        rv = 1.0 / jnp.maximum(cv, 1.0)
                        for g in range(8):
                            row = wbuf[r, pl.ds(g * L, L)] * rv
                            if relu:
                                row = jnp.maximum(row, 0.0)
                            wbuf[r, pl.ds(g * L, L)] = row
                        return c3

                    lax.fori_loop(0, wb, row_body, 0)
                    pltpu.sync_copy(wbuf, out_hbm.at[pl.ds(lo + b * wb, wb)])
                    return c2

                lax.fori_loop(0, (nblk_real - sid + NS - 1) // NS,
                              wb_body, 0)
                plsc.subcore_barrier()

            return qcarry

        lax.fori_loop(0, qmax, q_body, 0)

    scratch = [
        pltpu.VMEM_SHARED((racc, D), jnp.float32),
        pltpu.VMEM_SHARED((racc,), jnp.float32),
        pltpu.VMEM((CH,), jnp.int32),
        pltpu.VMEM((2, fh), jnp.int32),
    ]
    if rm:
        scratch.append(pltpu.VMEM((2, fh), jnp.int32))
    scratch += [
        pltpu.VMEM((2, fh), jnp.int32),
        pltpu.VMEM((fh, D), jnp.float32),
    ]
    if rm:
        scratch += [
            pltpu.VMEM((fh, 2 * D), jnp.float32),
            pltpu.VMEM((fh, D), jnp.float32),
        ]
    scratch += [
        pltpu.VMEM((wb, D), jnp.float32),
        pltpu.VMEM((wb,), jnp.float32),
        pltpu.VMEM((fh,), jnp.float32),
        pltpu.VMEM((L,), jnp.float32),
    ]
    if rm:
        scratch.append(pltpu.VMEM((L,), jnp.float32))
    scratch.append(pltpu.SemaphoreType.DMA)
    if rm:
        scratch.append(pltpu.SemaphoreType.DMA)

    return pl.kernel(
        body,
        out_type=_f32((npad, D)),
        mesh=_mesh(),
        compiler_params=_SC_PARAMS,
        scratch_types=scratch,
    )


def _make_gather_kernel(ns_pad, ne_pad, nb):
    def body(s_hbm, e_hbm, sid_hbm, eid_hbm, out_s, out_e,
             ibuf, rows_v, sem1):
        cid = lax.axis_index("c")
        sid = lax.axis_index("s")
        base = (sid * NC + cid) * (nb // NW)
        n = nb // NW
        pltpu.sync_copy(sid_hbm.at[pl.ds(base, n)], ibuf)
        pltpu.async_copy(s_hbm.at[ibuf], rows_v, sem1).wait()
        pltpu.sync_copy(rows_v, out_s.at[pl.ds(base, n)])
        pltpu.sync_copy(eid_hbm.at[pl.ds(base, n)], ibuf)
        pltpu.async_copy(e_hbm.at[ibuf], rows_v, sem1).wait()
        pltpu.sync_copy(rows_v, out_e.at[pl.ds(base, n)])

    return pl.kernel(
        body,
        out_type=(_f32((nb, D)), _f32((nb, D))),
        mesh=_mesh(),
        compiler_params=_SC_PARAMS,
        scratch_types=[
            pltpu.VMEM((nb // NW,), jnp.int32),
            pltpu.VMEM((nb // NW, D), jnp.float32),
            pltpu.SemaphoreType.DMA,
        ],
    )


def _tc_call(body, out_shape, *args):
    return pl.pallas_call(
        body, out_shape=jax.ShapeDtypeStruct(out_shape, jnp.float32))(*args)


def _prep_g(x, w):
    def body(x_ref, w_ref, o_ref):
        o_ref[...] = (x_ref[...] @ w_ref[...]) @ w_ref[...].T

    return _tc_call(body, (K_NUM, D), x, w)


def _make_t(x, w):
    def body(x_ref, w_ref, o_ref):
        xw = x_ref[...] @ w_ref[...]
        o_ref[...] = jnp.concatenate([xw @ w_ref[...].T, xw], axis=-1)

    return _tc_call(body, (K_NUM, 2 * D), x, w)


def _post_a(mpart, cpart, w):
    def body(mp_ref, cp_ref, w_ref, o_ref):
        p = mp_ref[0] + mp_ref[1]
        c = cp_ref[0, :K_NUM] + cp_ref[1, :K_NUM]
        scale = 1.0 / jnp.maximum(c, 1.0)
        o_ref[...] = (p @ w_ref[...]) * scale[:, None]

    return _tc_call(body, (K_NUM, D), mpart, cpart, w)


def _post_rev(mpart, cpart, relu):
    def body(mp_ref, cp_ref, o_ref):
        p = mp_ref[0] + mp_ref[1]
        c = cp_ref[0, :K_NUM] + cp_ref[1, :K_NUM]
        r = p * (1.0 / jnp.maximum(c, 1.0))[:, None]
        if relu:
            r = jnp.maximum(r, 0.0)
        o_ref[...] = r

    return _tc_call(body, (K_NUM, D), mpart, cpart)


def _head(kn_r, k1, k2, stu_v, exer_v, Wfc4, bfc4, Wfc5, bfc5, Wfc3, bfc3,
          Wfc7, bfc7):
    nb = kn_r.shape[0]
    blk = 512

    def body(knr_ref, k1_ref, k2_ref, sv_ref, ev_ref, w4_ref, b4_ref,
             w5_ref, b5_ref, w3_ref, b3_ref, w7_ref, b7_ref, o_ref):
        k = 0.5 * (k1_ref[...] + k2_ref[...])
        knr = knr_ref[...]
        kn_v = (knr @ k) / (jnp.sum(knr, axis=-1, keepdims=True) + 1e-8)
        xs = jnp.tanh(jnp.concatenate([sv_ref[...], kn_v], -1) @ w4_ref[...]
                      + b4_ref[...])
        xe = jnp.tanh(jnp.concatenate([ev_ref[...], kn_v], -1) @ w5_ref[...]
                      + b5_ref[...])
        h = jax.nn.relu((xs - xe) @ w3_ref[...] + b3_ref[...])
        o_ref[...] = jax.nn.sigmoid(h @ w7_ref[...] + b7_ref[...])

    full = lambda shape: pl.BlockSpec(shape, lambda i: (0,) * len(shape))
    return pl.pallas_call(
        body,
        out_shape=jax.ShapeDtypeStruct((nb, 1), jnp.float32),
        grid=(nb // blk,),
        in_specs=[
            pl.BlockSpec((blk, K_NUM), lambda i: (i, 0)),
            full((K_NUM, D)), full((K_NUM, D)),
            pl.BlockSpec((blk, D), lambda i: (i, 0)),
            pl.BlockSpec((blk, D), lambda i: (i, 0)),
            full(Wfc4.shape), full(bfc4.shape), full(Wfc5.shape),
            full(bfc5.shape), full(Wfc3.shape), full(bfc3.shape),
            full(Wfc7.shape), full(bfc7.shape),
        ],
        out_specs=pl.BlockSpec((blk, 1), lambda i: (i, 0)),
    )(kn_r, k1, k2, stu_v, exer_v, Wfc4, bfc4, Wfc5, bfc5, Wfc3, bfc3,
      Wfc7, bfc7)


def _side(x_table, kn_table, pidx, nnz, npad, rchunk, W1, W2,
          Wr1, Wr2, z2d, z1d, fh_f2=128):
    nx = x_table.shape[0]
    a_x = _make_a_kernel(nnz, nx, alpha=True)
    a_s = _make_a_kernel(nnz, npad, alpha=True)
    a_m = _make_a_kernel(nnz, npad, alpha=False)
    b_f2 = _make_b_kernel(nnz, npad, rchunk, "f2", relu=False, fh_f2=fh_f2)
    b_f2r = _make_b_kernel(nnz, npad, rchunk, "f2", relu=True, fh_f2=fh_f2)
    b_rm = _make_b_kernel(nnz, npad, rchunk, "rm", relu=False)

    hg1 = _prep_g(kn_table, W1)
    mp, cp = a_x(x_table, hg1, pidx, z2d, z1d)
    m1 = _post_a(mp, cp, W1)
    s1 = b_f2(m1, pidx, z2d, z1d)

    hg2 = _prep_g(kn_table, W2)
    mp, cp = a_s(s1, hg2, pidx, z2d, z1d)
    m2 = _post_a(mp, cp, W2)
    s = b_f2r(m2, pidx, z2d, z1d)

    t3 = _make_t(kn_table, Wr1)
    m3 = b_rm(s, t3, pidx, z2d, z1d)
    mp, cp = a_m(m3, pidx, z2d, z1d)
    k1a = _post_rev(mp, cp, relu=False)

    t4 = _make_t(k1a, Wr2)
    m4 = b_rm(s, t4, pidx, z2d, z1d)
    mp, cp = a_m(m4, pidx, z2d, z1d)
    k1 = _post_rev(mp, cp, relu=True)
    return s, k1


def kernel(stu_id, exer_id, kn_r, hidx_sk, hidx_ek, stu_table, exer_table,
           kn_table, Ws1, Ws2, Wks1, Wks2, We1, We2, Wke1, Wke2,
           Wfc4, bfc4, Wfc5, bfc5, Wfc3, bfc3, Wfc7, bfc7):
    z2d = jnp.zeros((CH, D), jnp.float32)
    z1d = jnp.zeros((CH,), jnp.float32)

    p_sk = (hidx_sk[0].astype(jnp.int32) * 2048 +
            hidx_sk[1].astype(jnp.int32))
    p_ek = (hidx_ek[0].astype(jnp.int32) * 2048 +
            hidx_ek[1].astype(jnp.int32))

    s, k1 = _side(stu_table, kn_table, p_sk, 400000, 58368, 9728,
                  Ws1, Ws2, Wks1, Wks2, z2d, z1d)
    e, k2 = _side(exer_table, kn_table, p_ek, 160000, 20224, 10112,
                  We1, We2, Wke1, Wke2, z2d, z1d, fh_f2=64)

    nb = stu_id.shape[0]
    gk = _make_gather_kernel(58368, 20224, nb)
    stu_v, exer_v = gk(s, e, stu_id.astype(jnp.int32),
                       exer_id.astype(jnp.int32))

    return _head(kn_r, k1, k2, stu_v, exer_v, Wfc4, bfc4, Wfc5, bfc5,
                 Wfc3, bfc3, Wfc7, bfc7)


# speedup vs baseline: 2.3413x; 1.0636x over previous
import functools
import math

import jax
import jax.numpy as jnp
from jax import lax
from jax.experimental import pallas as pl
from jax.experimental.pallas import tpu as pltpu
from jax.experimental.pallas import tpu_sc as plsc

NC = 2
NS = 16
NW = NC * NS
L = 16
D = 128
CH = 128
K_NUM = 2000
KPAD = 2048
RSQ = 1.0 / math.sqrt(D)

_mesh = functools.partial(
    plsc.VectorSubcoreMesh, core_axis_name="c", subcore_axis_name="s",
    num_cores=NC, num_subcores=NS)
_SC_PARAMS = pltpu.CompilerParams(needs_layout_passes=False)


def _f32(shape):
    return jax.ShapeDtypeStruct(shape, jnp.float32)


def _fill_ones(ref):
    for i in range(ref.shape[0] // L):
        ref[pl.ds(i * L, L)] = jnp.full((L,), 1.0, jnp.float32)


def _hsum_splat(tmp_ref, v):
    lanes = lax.iota(jnp.int32, L)
    for sh in (8, 4, 2, 1):
        tmp_ref[...] = v
        v = v + plsc.load_gather(tmp_ref, [lanes ^ sh])
    return v


def _split2k(sid, copy_fn):
    @pl.when(sid < 15)
    def _():
        copy_fn(sid * CH, CH)

    @pl.when(sid == 15)
    def _():
        copy_fn(15 * CH, 80)


def _make_a_kernel(nnz, nx, alpha):
    def body(*refs):
        if alpha:
            (x_hbm, g_hbm, pidx, z2d, z1d, mpart, cpart,
             m_sp, cnt_sp, pbuf, nbuf0, ebuf0, xbuf0, gbuf0,
             nbuf1, ebuf1, xbuf1, gbuf1, abuf, ones_v,
             tmp16, tmp16b, semx0, semx1, semg0, semg1) = refs
            nbufs, ebufs = (nbuf0, nbuf1), (ebuf0, ebuf1)
            xbufs, gbufs = (xbuf0, xbuf1), (gbuf0, gbuf1)
            semxs, semgs = (semx0, semx1), (semg0, semg1)
        else:
            (x_hbm, pidx, z2d, z1d, mpart, cpart,
             m_sp, cnt_sp, pbuf, nbuf0, ebuf0, xbuf0,
             nbuf1, ebuf1, xbuf1, ones_v, semx0, semx1) = refs
            nbufs, ebufs = (nbuf0, nbuf1), (ebuf0, ebuf1)
            xbufs = (xbuf0, xbuf1)
            semxs = (semx0, semx1)
        cid = lax.axis_index("c")
        sid = lax.axis_index("s")
        w = sid * NC + cid

        def _zm(off, sz):
            pltpu.sync_copy(z2d.at[pl.ds(0, sz)], m_sp.at[pl.ds(off, sz)])

        _split2k(sid, _zm)
        pltpu.sync_copy(z1d, cnt_sp.at[pl.ds(sid * CH, CH)])
        _fill_ones(ones_v)
        plsc.subcore_barrier()

        nchunks = nnz // CH
        trips = (nchunks - w + NW - 1) // NW

        def issue(j, slot):
            base = (w + j * NW) * CH
            pltpu.sync_copy(pidx.at[pl.ds(base, CH)], pbuf)
            for v in range(8):
                pv = pbuf[pl.ds(v * L, L)]
                nbufs[slot][pl.ds(v * L, L)] = pv >> 11
                ebufs[slot][pl.ds(v * L, L)] = pv & 2047
            pltpu.async_copy(x_hbm.at[nbufs[slot]], xbufs[slot], semxs[slot])
            if alpha:
                pltpu.async_copy(g_hbm.at[ebufs[slot]], gbufs[slot],
                                 semgs[slot])

        def process(slot):
            xb = xbufs[slot]
            pltpu.make_async_copy(x_hbm.at[nbufs[slot]], xb,
                                  semxs[slot]).wait()
            if alpha:
                gb = gbufs[slot]
                pltpu.make_async_copy(g_hbm.at[ebufs[slot]], gb,
                                      semgs[slot]).wait()

                def row_body(r2, c2):
                    for half, tref in ((0, tmp16), (1, tmp16b)):
                        r = r2 * 2 + half
                        acc = jnp.zeros((L,), jnp.float32)
                        xr = []
                        for g in range(8):
                            xv = xb[r, pl.ds(g * L, L)]
                            xr.append(xv)
                            acc = acc + xv * gb[r, pl.ds(g * L, L)]
                        z = _hsum_splat(tref, acc)
                        av = 1.0 / (1.0 + jnp.exp(-z * RSQ))
                        for g in range(8):
                            abuf[r, pl.ds(g * L, L)] = xr[g] * av
                    return c2

                lax.fori_loop(0, CH // 2, row_body, 0)
                src_buf = abuf
            else:
                src_buf = xb
            pltpu.sync_copy(src_buf, m_sp.at[ebufs[slot]], add=True)
            pltpu.sync_copy(ones_v, cnt_sp.at[ebufs[slot]], add=True)

        @pl.when(trips > 0)
        def _prologue():
            issue(0, 0)

        def pair_body(j2, carry):
            j0 = j2 * 2

            @pl.when(j0 + 1 < trips)
            def _():
                issue(j0 + 1, 1)

            process(0)

            @pl.when(j0 + 1 < trips)
            def _():
                @pl.when(j0 + 2 < trips)
                def _():
                    issue(j0 + 2, 0)

                process(1)

            return carry

        lax.fori_loop(0, (trips + 1) // 2, pair_body, 0)
        plsc.subcore_barrier()

        def _wm(off, sz):
            pltpu.sync_copy(m_sp.at[pl.ds(off, sz)],
                            mpart.at[cid, pl.ds(off, sz)])

        _split2k(sid, _wm)
        pltpu.sync_copy(cnt_sp.at[pl.ds(sid * CH, CH)],
                        cpart.at[cid, pl.ds(sid * CH, CH)])

    scratch = [
        pltpu.VMEM_SHARED((K_NUM, D), jnp.float32),
        pltpu.VMEM_SHARED((KPAD,), jnp.float32),
        pltpu.VMEM((CH,), jnp.int32),
    ]
    if alpha:
        scratch += [
            pltpu.VMEM((CH,), jnp.int32),
            pltpu.VMEM((CH,), jnp.int32),
            pltpu.VMEM((CH, D), jnp.float32),
            pltpu.VMEM((CH, D), jnp.float32),
            pltpu.VMEM((CH,), jnp.int32),
            pltpu.VMEM((CH,), jnp.int32),
            pltpu.VMEM((CH, D), jnp.float32),
            pltpu.VMEM((CH, D), jnp.float32),
            pltpu.VMEM((CH, D), jnp.float32),
            pltpu.VMEM((CH,), jnp.float32),
            pltpu.VMEM((L,), jnp.float32),
            pltpu.VMEM((L,), jnp.float32),
            pltpu.SemaphoreType.DMA,
            pltpu.SemaphoreType.DMA,
            pltpu.SemaphoreType.DMA,
            pltpu.SemaphoreType.DMA,
        ]
    else:
        scratch += [
            pltpu.VMEM((CH,), jnp.int32),
            pltpu.VMEM((CH,), jnp.int32),
            pltpu.VMEM((CH, D), jnp.float32),
            pltpu.VMEM((CH,), jnp.int32),
            pltpu.VMEM((CH,), jnp.int32),
            pltpu.VMEM((CH, D), jnp.float32),
            pltpu.VMEM((CH,), jnp.float32),
            pltpu.SemaphoreType.DMA,
            pltpu.SemaphoreType.DMA,
        ]

    return pl.kernel(
        body,
        out_type=(_f32((NC, K_NUM, D)), _f32((NC, KPAD))),
        mesh=_mesh(),
        compiler_params=_SC_PARAMS,
        scratch_types=scratch,
    )


def _make_b_kernel(nnz, npad, rchunk, mode, relu, fh_f2=128):
    rr = rchunk
    racc = rr + CH
    nch = npad // rr
    qmax = (nch + NC - 1) // NC
    rm = mode == "rm"
    fh = 64 if rm else fh_f2
    wb = 64 if rm else 128
    fmask = 2 * fh - 1
    fshift = fh.bit_length() - 1

    def body(*refs):
        if rm:
            (s_hbm, t_hbm, pidx, z2d, z1d, out_hbm,
             acc_sp, cnt_sp, pbuf, pbuf1, semp0, semp1,
             pend_nl, pend_ng, pend_e,
             rows_v, trows_v, abuf, wbuf, cbuf, ones_v, tmp16, tmp16b,
             sem1, sem2) = refs
        else:
            (m_hbm, pidx, z2d, z1d, out_hbm,
             acc_sp, cnt_sp, pbuf, pbuf1, semp0, semp1,
             pend_nl, pend_e,
             rows_v, wbuf, cbuf, ones_v, tmp16, sem1) = refs
        cid = lax.axis_index("c")
        sid = lax.axis_index("s")
        small_hbm = t_hbm if rm else m_hbm
        _fill_ones(ones_v)

        def flush(par):
            idx_e = pend_e.at[par]
            idx_nl = pend_nl.at[par]
            if rm:
                cp1 = pltpu.async_copy(s_hbm.at[pend_ng.at[par]], rows_v,
                                       sem1)
                cp2 = pltpu.async_copy(t_hbm.at[idx_e], trows_v, sem2)
                cp1.wait()
                cp2.wait()

                def row_body(r2, c2):
                    for half, tref in ((0, tmp16), (1, tmp16b)):
                        r = r2 * 2 + half
                        acc = jnp.zeros((L,), jnp.float32)
                        tw = []
                        for g in range(8):
                            acc = acc + (rows_v[r, pl.ds(g * L, L)] *
                                         trows_v[r, pl.ds(g * L, L)])
                            tw.append(trows_v[r, pl.ds(D + g * L, L)])
                        z = _hsum_splat(tref, acc)
                        av = 1.0 / (1.0 + jnp.exp(-z * RSQ))
                        for g in range(8):
                            abuf[r, pl.ds(g * L, L)] = tw[g] * av
                    return c2

                lax.fori_loop(0, fh // 2, row_body, 0)
                pltpu.sync_copy(abuf, acc_sp.at[idx_nl], add=True)
            else:
                cp = pltpu.async_copy(m_hbm.at[idx_e], rows_v, sem1)
                cp.wait()
                pltpu.sync_copy(rows_v, acc_sp.at[idx_nl], add=True)
            pltpu.sync_copy(ones_v, cnt_sp.at[idx_nl], add=True)

        nblk_acc = racc // CH
        nblk_real = rr // wb
        scan_chunks = nnz // CH

        def q_body(q, qcarry):
            ch = cid + NC * q

            @pl.when(ch < nch)
            def _do_chunk():
                lo = ch * rr
                hi = lo + rr

                def zero_body(bj, c2):
                    b = sid + bj * NS
                    pltpu.sync_copy(z2d, acc_sp.at[pl.ds(b * CH, CH)])
                    pltpu.sync_copy(z1d, cnt_sp.at[pl.ds(b * CH, CH)])
                    return c2

                lax.fori_loop(0, (nblk_acc - sid + NS - 1) // NS,
                              zero_body, 0)
                plsc.subcore_barrier()

                def process8(pb, pc, fl):
                    for v in range(8):
                        pv = pb[pl.ds(v * L, L)]
                        nv = pv >> 11
                        ev = pv & 2047
                        within = (nv >= lo) & (nv < hi)
                        wi = within.astype(jnp.int32)
                        cs = plsc.cumsum(wi)
                        pos = (jnp.full((L,), pc, jnp.int32) + cs - 1) & fmask
                        hi_i = pos >> fshift
                        lo_i = pos & (fh - 1)
                        plsc.store_scatter(pend_nl, [hi_i, lo_i], nv - lo,
                                           mask=within)
                        if rm:
                            plsc.store_scatter(pend_ng, [hi_i, lo_i], nv,
                                               mask=within)
                        plsc.store_scatter(pend_e, [hi_i, lo_i], ev,
                                           mask=within)
                        pc = pc + cs[L - 1]
                        do = (pc - fl) >= fh

                        @pl.when(do)
                        def _():
                            flush((fl // fh) & 1)

                        fl = jnp.where(do, fl + fh, fl)
                    return pc, fl

                strips = scan_chunks // NS

                def issue_scan(j, pb, sem):
                    base = (sid + j * NS) * CH
                    return pltpu.async_copy(
                        pidx.at[pl.ds(base, CH)], pb, sem)

                def wait_scan(j, pb, sem):
                    base = (sid + j * NS) * CH
                    pltpu.make_async_copy(
                        pidx.at[pl.ds(base, CH)], pb, sem).wait()

                issue_scan(0, pbuf, semp0)

                def scan_pair(j2, carry):
                    pc, fl = carry
                    j0 = 2 * j2
                    issue_scan(j0 + 1, pbuf1, semp1)
                    wait_scan(j0, pbuf, semp0)
                    pc, fl = process8(pbuf, pc, fl)

                    @pl.when(j0 + 2 < strips)
                    def _():
                        issue_scan(j0 + 2, pbuf, semp0)

                    wait_scan(j0 + 1, pbuf1, semp1)
                    pc, fl = process8(pbuf1, pc, fl)
                    return pc, fl

                pc, fl = lax.fori_loop(0, strips // 2, scan_pair,
                                       (jnp.int32(0), jnp.int32(0)))

                @pl.when(pc > fl)
                def _drain():
                    lanes = lax.iota(jnp.int32, L)
                    for v in range(fh // L):
                        lanepos = jnp.full((L,), pc, jnp.int32) + lanes + v * L
                        mask = lanepos < fl + fh
                        pos = lanepos & fmask
                        hi_i = pos >> fshift
                        lo_i = pos & (fh - 1)
                        plsc.store_scatter(
                            pend_nl, [hi_i, lo_i],
                            jnp.full((L,), rr, jnp.int32), mask=mask)
                        if rm:
                            plsc.store_scatter(
                                pend_ng, [hi_i, lo_i],
                                jnp.zeros((L,), jnp.int32), mask=mask)
                        plsc.store_scatter(
                            pend_e, [hi_i, lo_i],
                            jnp.zeros((L,), jnp.int32), mask=mask)
                    flush((fl // fh) & 1)

                plsc.subcore_barrier()

                def wb_body(bj, c2):
                    b = sid + bj * NS
                    pltpu.sync_copy(acc_sp.at[pl.ds(b * wb, wb)], wbuf)
                    pltpu.sync_copy(cnt_sp.at[pl.ds(b * wb, wb)], cbuf)

                    def row_body(r, c3):
                        cv = plsc.load_gather(
                            cbuf, [jnp.full((L,), r, jnp.int32)])
                        rv = 1.0 / jnp.maximum(cv, 1.0)
                        for g in range(8):
                            row = wbuf[r, pl.ds(g * L, L)] * rv
                            if relu:
                                row = jnp.maximum(row, 0.0)
                            wbuf[r, pl.ds(g * L, L)] = row
                        return c3

                    lax.fori_loop(0, wb, row_body, 0)
                    pltpu.sync_copy(wbuf, out_hbm.at[pl.ds(lo + b * wb, wb)])
                    return c2

                lax.fori_loop(0, (nblk_real - sid + NS - 1) // NS,
                              wb_body, 0)
                plsc.subcore_barrier()

            return qcarry

        lax.fori_loop(0, qmax, q_body, 0)

    scratch = [
        pltpu.VMEM_SHARED((racc, D), jnp.float32),
        pltpu.VMEM_SHARED((racc,), jnp.float32),
        pltpu.VMEM((CH,), jnp.int32),
        pltpu.VMEM((CH,), jnp.int32),
        pltpu.SemaphoreType.DMA,
        pltpu.SemaphoreType.DMA,
        pltpu.VMEM((2, fh), jnp.int32),
    ]
    if rm:
        scratch.append(pltpu.VMEM((2, fh), jnp.int32))
    scratch += [
        pltpu.VMEM((2, fh), jnp.int32),
        pltpu.VMEM((fh, D), jnp.float32),
    ]
    if rm:
        scratch += [
            pltpu.VMEM((fh, 2 * D), jnp.float32),
            pltpu.VMEM((fh, D), jnp.float32),
        ]
    scratch += [
        pltpu.VMEM((wb, D), jnp.float32),
        pltpu.VMEM((wb,), jnp.float32),
        pltpu.VMEM((fh,), jnp.float32),
        pltpu.VMEM((L,), jnp.float32),
    ]
    if rm:
        scratch.append(pltpu.VMEM((L,), jnp.float32))
    scratch.append(pltpu.SemaphoreType.DMA)
    if rm:
        scratch.append(pltpu.SemaphoreType.DMA)

    return pl.kernel(
        body,
        out_type=_f32((npad, D)),
        mesh=_mesh(),
        compiler_params=_SC_PARAMS,
        scratch_types=scratch,
    )


def _make_gather_kernel(ns_pad, ne_pad, nb):
    def body(s_hbm, e_hbm, sid_hbm, eid_hbm, out_s, out_e,
             ibuf, rows_v, sem1):
        cid = lax.axis_index("c")
        sid = lax.axis_index("s")
        base = (sid * NC + cid) * (nb // NW)
        n = nb // NW
        pltpu.sync_copy(sid_hbm.at[pl.ds(base, n)], ibuf)
        pltpu.async_copy(s_hbm.at[ibuf], rows_v, sem1).wait()
        pltpu.sync_copy(rows_v, out_s.at[pl.ds(base, n)])
        pltpu.sync_copy(eid_hbm.at[pl.ds(base, n)], ibuf)
        pltpu.async_copy(e_hbm.at[ibuf], rows_v, sem1).wait()
        pltpu.sync_copy(rows_v, out_e.at[pl.ds(base, n)])

    return pl.kernel(
        body,
        out_type=(_f32((nb, D)), _f32((nb, D))),
        mesh=_mesh(),
        compiler_params=_SC_PARAMS,
        scratch_types=[
            pltpu.VMEM((nb // NW,), jnp.int32),
            pltpu.VMEM((nb // NW, D), jnp.float32),
            pltpu.SemaphoreType.DMA,
        ],
    )


def _tc_call(body, out_shape, *args):
    return pl.pallas_call(
        body, out_shape=jax.ShapeDtypeStruct(out_shape, jnp.float32))(*args)


def _prep_g(x, w):
    def body(x_ref, w_ref, o_ref):
        o_ref[...] = (x_ref[...] @ w_ref[...]) @ w_ref[...].T

    return _tc_call(body, (K_NUM, D), x, w)


def _make_t(x, w):
    def body(x_ref, w_ref, o_ref):
        xw = x_ref[...] @ w_ref[...]
        o_ref[...] = jnp.concatenate([xw @ w_ref[...].T, xw], axis=-1)

    return _tc_call(body, (K_NUM, 2 * D), x, w)


def _post_a(mpart, cpart, w):
    def body(mp_ref, cp_ref, w_ref, o_ref):
        p = mp_ref[0] + mp_ref[1]
        c = cp_ref[0, :K_NUM] + cp_ref[1, :K_NUM]
        scale = 1.0 / jnp.maximum(c, 1.0)
        o_ref[...] = (p @ w_ref[...]) * scale[:, None]

    return _tc_call(body, (K_NUM, D), mpart, cpart, w)


def _post_rev(mpart, cpart, relu):
    def body(mp_ref, cp_ref, o_ref):
        p = mp_ref[0] + mp_ref[1]
        c = cp_ref[0, :K_NUM] + cp_ref[1, :K_NUM]
        r = p * (1.0 / jnp.maximum(c, 1.0))[:, None]
        if relu:
            r = jnp.maximum(r, 0.0)
        o_ref[...] = r

    return _tc_call(body, (K_NUM, D), mpart, cpart)


def _head(kn_r, k1, k2, stu_v, exer_v, Wfc4, bfc4, Wfc5, bfc5, Wfc3, bfc3,
          Wfc7, bfc7):
    nb = kn_r.shape[0]
    blk = 512

    def body(knr_ref, k1_ref, k2_ref, sv_ref, ev_ref, w4_ref, b4_ref,
             w5_ref, b5_ref, w3_ref, b3_ref, w7_ref, b7_ref, o_ref):
        k = 0.5 * (k1_ref[...] + k2_ref[...])
        knr = knr_ref[...]
        kn_v = (knr @ k) / (jnp.sum(knr, axis=-1, keepdims=True) + 1e-8)
        xs = jnp.tanh(jnp.concatenate([sv_ref[...], kn_v], -1) @ w4_ref[...]
                      + b4_ref[...])
        xe = jnp.tanh(jnp.concatenate([ev_ref[...], kn_v], -1) @ w5_ref[...]
                      + b5_ref[...])
        h = jax.nn.relu((xs - xe) @ w3_ref[...] + b3_ref[...])
        o_ref[...] = jax.nn.sigmoid(h @ w7_ref[...] + b7_ref[...])

    full = lambda shape: pl.BlockSpec(shape, lambda i: (0,) * len(shape))
    return pl.pallas_call(
        body,
        out_shape=jax.ShapeDtypeStruct((nb, 1), jnp.float32),
        grid=(nb // blk,),
        in_specs=[
            pl.BlockSpec((blk, K_NUM), lambda i: (i, 0)),
            full((K_NUM, D)), full((K_NUM, D)),
            pl.BlockSpec((blk, D), lambda i: (i, 0)),
            pl.BlockSpec((blk, D), lambda i: (i, 0)),
            full(Wfc4.shape), full(bfc4.shape), full(Wfc5.shape),
            full(bfc5.shape), full(Wfc3.shape), full(bfc3.shape),
            full(Wfc7.shape), full(bfc7.shape),
        ],
        out_specs=pl.BlockSpec((blk, 1), lambda i: (i, 0)),
    )(kn_r, k1, k2, stu_v, exer_v, Wfc4, bfc4, Wfc5, bfc5, Wfc3, bfc3,
      Wfc7, bfc7)


def _side(x_table, kn_table, pidx, pidx_pad, nnz, npad, rchunk, W1, W2,
          Wr1, Wr2, z2d, z1d, fh_f2=128):
    nx = x_table.shape[0]
    nnz_pad = pidx_pad.shape[0]
    a_x = _make_a_kernel(nnz, nx, alpha=True)
    a_s = _make_a_kernel(nnz, npad, alpha=True)
    a_m = _make_a_kernel(nnz, npad, alpha=False)
    b_f2 = _make_b_kernel(nnz_pad, npad, rchunk, "f2", relu=False,
                          fh_f2=fh_f2)
    b_f2r = _make_b_kernel(nnz_pad, npad, rchunk, "f2", relu=True,
                           fh_f2=fh_f2)
    b_rm = _make_b_kernel(nnz_pad, npad, rchunk, "rm", relu=False)

    hg1 = _prep_g(kn_table, W1)
    mp, cp = a_x(x_table, hg1, pidx, z2d, z1d)
    m1 = _post_a(mp, cp, W1)
    s1 = b_f2(m1, pidx_pad, z2d, z1d)

    hg2 = _prep_g(kn_table, W2)
    mp, cp = a_s(s1, hg2, pidx, z2d, z1d)
    m2 = _post_a(mp, cp, W2)
    s = b_f2r(m2, pidx_pad, z2d, z1d)

    t3 = _make_t(kn_table, Wr1)
    m3 = b_rm(s, t3, pidx_pad, z2d, z1d)
    mp, cp = a_m(m3, pidx, z2d, z1d)
    k1a = _post_rev(mp, cp, relu=False)

    t4 = _make_t(k1a, Wr2)
    m4 = b_rm(s, t4, pidx_pad, z2d, z1d)
    mp, cp = a_m(m4, pidx, z2d, z1d)
    k1 = _post_rev(mp, cp, relu=True)
    return s, k1


def kernel(stu_id, exer_id, kn_r, hidx_sk, hidx_ek, stu_table, exer_table,
           kn_table, Ws1, Ws2, Wks1, Wks2, We1, We2, Wke1, Wke2,
           Wfc4, bfc4, Wfc5, bfc5, Wfc3, bfc3, Wfc7, bfc7):
    z2d = jnp.zeros((CH, D), jnp.float32)
    z1d = jnp.zeros((CH,), jnp.float32)

    p_sk = (hidx_sk[0].astype(jnp.int32) * 2048 +
            hidx_sk[1].astype(jnp.int32))
    p_ek = (hidx_ek[0].astype(jnp.int32) * 2048 +
            hidx_ek[1].astype(jnp.int32))

    padv = jnp.int32(1 << 30)
    p_sk_pad = jnp.concatenate(
        [p_sk, jnp.full((401408 - 400000,), padv)])
    p_ek_pad = jnp.concatenate(
        [p_ek, jnp.full((163840 - 160000,), padv)])

    s, k1 = _side(stu_table, kn_table, p_sk, p_sk_pad, 400000, 58368, 9728,
                  Ws1, Ws2, Wks1, Wks2, z2d, z1d)
    e, k2 = _side(exer_table, kn_table, p_ek, p_ek_pad, 160000, 20224,
                  10112, We1, We2, Wke1, Wke2, z2d, z1d, fh_f2=64)

    nb = stu_id.shape[0]
    gk = _make_gather_kernel(58368, 20224, nb)
    stu_v, exer_v = gk(s, e, stu_id.astype(jnp.int32),
                       exer_id.astype(jnp.int32))

    return _head(kn_r, k1, k2, stu_v, exer_v, Wfc4, bfc4, Wfc5, bfc5,
                 Wfc3, bfc3, Wfc7, bfc7)
